# baseline (speedup 1.0000x reference)
import functools

import jax
import jax.numpy as jnp
import numpy as np
from jax.experimental import pallas as pl
from jax.experimental.pallas import tpu as pltpu

_TASK_ID = 5
_NUM_TASKS = 10
_GRP = 32
_SIZE = 32


def _shift_zf(a, k):
    n, w = a.shape
    if k == 0:
        return a
    z = jnp.zeros((abs(k), w), a.dtype)
    if k > 0:
        return jnp.concatenate([a[k:], z], axis=0)
    return jnp.concatenate([z, a[:n + k]], axis=0)


def _conv_banded(src, mjm, mjp, w_bd, width, nb, cin8):
    planes = {-1: _shift_rows_masked(src, -1, mjm),
              0: src,
              1: _shift_rows_masked(src, 1, mjp)}
    parts = []
    for b in range(nb):
        sl = [planes[dj][:, b * cin8:(b + 1) * cin8] for dj in (-1, 0, 1)]
        cat = jnp.concatenate(
            [_shift_zf(sl[dj + 1], di * width)
             for di in (-1, 0, 1) for dj in (-1, 0, 1)], axis=1)
        parts.append(jnp.dot(cat, w_bd, preferred_element_type=jnp.float32))
    return jnp.concatenate(parts, axis=1) if nb > 1 else parts[0]


def _shift_rows_masked(a, k, mask):
    n = a.shape[0]
    k = k % n
    return jnp.concatenate([a[k:], a[:k]], axis=0) * mask


def _feat_kernel(x_ref, w1_ref, b1_ref, m1m_ref, m1p_ref,
                 w2_ref, b2_ref, m2m_ref, m2p_ref, o_ref, *, grp):
    nb = grp // 8
    c1 = 16 * grp
    c2 = 32 * grp
    x = x_ref[...]

    y1 = _conv_banded(x, m1m_ref[...], m1p_ref[...], w1_ref[...],
                      32, nb, 24)
    y1 = jnp.maximum(y1 + b1_ref[...], 0.0)

    a = y1.reshape(512, 2, c1)
    m = jnp.maximum(a[:, 0, :], a[:, 1, :])
    a = m.reshape(16, 2, 16, c1)
    h1 = jnp.maximum(a[:, 0, :, :], a[:, 1, :, :]).reshape(256, c1)

    y2 = _conv_banded(h1, m2m_ref[...], m2p_ref[...], w2_ref[...],
                      16, nb, 128)
    y2 = y2 + b2_ref[...]

    a = y2.reshape(128, 2, c2)
    m = jnp.maximum(a[:, 0, :], a[:, 1, :])
    a = m.reshape(8, 2, 8, c2)
    h2 = jnp.maximum(a[:, 0, :, :], a[:, 1, :, :]).reshape(64, c2)
    o_ref[...] = jnp.transpose(h2)


def _head_kernel(h_ref, fw_ref, fb_ref, hw_ref, hb_ref, *o_refs):
    z = jnp.dot(h_ref[...], fw_ref[...], preferred_element_type=jnp.float32,
                 precision=jax.lax.Precision.HIGHEST)
    z = jnp.maximum(z + fb_ref[...], 0.0)
    y = jnp.dot(z, hw_ref[...],
                preferred_element_type=jnp.float32,
                precision=jax.lax.Precision.HIGHEST) + hb_ref[...]
    n = y.shape[1] // len(o_refs)
    for t, o_ref in enumerate(o_refs):
        o_ref[...] = y[:, t * n:(t + 1) * n]


def _col_mask(hw, width, lanes, dj):
    j = np.arange(hw) % width
    valid = (j >= 1) if dj < 0 else (j < width - 1)
    return np.repeat(valid[:, None], lanes, axis=1).astype(np.float32)


def _block_diag_taps(w, grp):
    t, cin, cout = w.shape
    eye = jnp.eye(grp, dtype=w.dtype)
    bd = eye[None, :, None, :, None] * w[:, None, :, None, :]
    return bd.reshape(t * grp * cin, grp * cout)


def kernel(x, c1_w, c1_b, c2_w, c2_b, fc1_w, fc1_b, head_w, head_b,
           scale1, shift1, scale2, shift2, scale3, shift3):
    bsz = x.shape[0]
    grp = _GRP
    ngrp = bsz // grp
    size = _SIZE
    npix = size * size

    s1 = scale1[_TASK_ID][None, :]
    sh1 = shift1[_TASK_ID][None, :]
    s2 = scale2[_TASK_ID][None, :]
    sh2 = shift2[_TASK_ID][None, :]
    s3 = scale3[_TASK_ID][None, :]
    sh3 = shift3[_TASK_ID][None, :]

    w1bd = _block_diag_taps(c1_w * s1[0][None, None, :], 8)
    b1 = jnp.tile(s1 * c1_b + sh1, (1, grp))
    w2bd = _block_diag_taps(c2_w * s2[0][None, None, :], 8)
    b2 = jnp.tile(s2 * c2_b + sh2, (1, grp))
    fw = (fc1_w * s3).reshape(64, 32, tdim_fc := fc1_w.shape[1])
    fw = fw.transpose(1, 0, 2).reshape(64 * 32, tdim_fc)
    fb = s3 * fc1_b + sh3

    m1m = _col_mask(npix, size, 3 * grp, -1)
    m1p = _col_mask(npix, size, 3 * grp, 1)
    m2m = _col_mask(npix // 4, size // 2, 16 * grp, -1)
    m2p = _col_mask(npix // 4, size // 2, 16 * grp, 1)

    xg = x.reshape(ngrp, grp, 3, npix).transpose(0, 3, 1, 2)
    xg = xg.reshape(ngrp * npix, grp * 3)

    const = lambda shape: pl.BlockSpec(shape, lambda g: (0, 0))
    h2 = pl.pallas_call(
        functools.partial(_feat_kernel, grp=grp),
        out_shape=jax.ShapeDtypeStruct((ngrp * 32 * grp, 64), jnp.float32),
        grid=(ngrp,),
        in_specs=[
            pl.BlockSpec((npix, grp * 3), lambda g: (g, 0)),
            const(w1bd.shape), const(b1.shape),
            const(m1m.shape), const(m1p.shape),
            const(w2bd.shape), const(b2.shape),
            const(m2m.shape), const(m2p.shape),
        ],
        out_specs=pl.BlockSpec((32 * grp, 64), lambda g: (g, 0)),
        compiler_params=pltpu.CompilerParams(
            dimension_semantics=("parallel",)),
    )(xg, w1bd, b1, m1m, m1p, w2bd, b2, m2m, m2p)

    hflat = h2.reshape(bsz, 64 * 32)

    bm = 256 if bsz % 256 == 0 else bsz
    tdim = head_w.shape[1]
    n = tdim // _NUM_TASKS
    outs = pl.pallas_call(
        _head_kernel,
        out_shape=[jax.ShapeDtypeStruct((bsz, n), jnp.float32)
                   for _ in range(_NUM_TASKS)],
        grid=(bsz // bm,),
        in_specs=[
            pl.BlockSpec((bm, fw.shape[0]), lambda g: (g, 0)),
            const(fw.shape), const(fb.shape),
            const(head_w.shape), const(head_b.shape),
        ],
        out_specs=[pl.BlockSpec((bm, n), lambda g: (g, 0))
                   for _ in range(_NUM_TASKS)],
        compiler_params=pltpu.CompilerParams(
            dimension_semantics=("parallel",)),
    )(hflat, fw, fb, head_w, head_b)
    return list(outs)


# speedup vs baseline: 1.1328x; 1.1328x over previous
import functools

import jax
import jax.numpy as jnp
import numpy as np
from jax.experimental import pallas as pl
from jax.experimental.pallas import tpu as pltpu

_TASK_ID = 5
_NUM_TASKS = 10
_GRP = 128
_SIZE = 32


def _shift_zf(a, k):
    n, w = a.shape
    if k == 0:
        return a
    z = jnp.zeros((abs(k), w), a.dtype)
    if k > 0:
        return jnp.concatenate([a[k:], z], axis=0)
    return jnp.concatenate([z, a[:n + k]], axis=0)


def _conv_banded(src, mjm, mjp, w_bd, width, nb, cin8):
    planes = {-1: _shift_rows_masked(src, -1, mjm),
              0: src,
              1: _shift_rows_masked(src, 1, mjp)}
    parts = []
    for b in range(nb):
        sl = [planes[dj][:, b * cin8:(b + 1) * cin8] for dj in (-1, 0, 1)]
        cat = jnp.concatenate(
            [_shift_zf(sl[dj + 1], di * width)
             for di in (-1, 0, 1) for dj in (-1, 0, 1)], axis=1)
        parts.append(jnp.dot(cat, w_bd, preferred_element_type=jnp.float32))
    return jnp.concatenate(parts, axis=1) if nb > 1 else parts[0]


def _shift_rows_masked(a, k, mask):
    n = a.shape[0]
    k = k % n
    return jnp.concatenate([a[k:], a[:k]], axis=0) * mask


def _feat_kernel(x_ref, w1_ref, b1_ref, m1m_ref, m1p_ref,
                 w2_ref, b2_ref, m2m_ref, m2p_ref, o_ref, *, grp):
    nb = grp // 8
    c1 = 16 * grp
    c2 = 32 * grp
    x = x_ref[...]

    y1 = _conv_banded(x, m1m_ref[...], m1p_ref[...], w1_ref[...],
                      32, nb, 24)
    y1 = jnp.maximum(y1 + b1_ref[...], 0.0)

    a = y1.reshape(512, 2, c1)
    m = jnp.maximum(a[:, 0, :], a[:, 1, :])
    a = m.reshape(16, 2, 16, c1)
    h1 = jnp.maximum(a[:, 0, :, :], a[:, 1, :, :]).reshape(256, c1)

    y2 = _conv_banded(h1, m2m_ref[...], m2p_ref[...], w2_ref[...],
                      16, nb, 128)
    y2 = y2 + b2_ref[...]

    a = y2.reshape(128, 2, c2)
    m = jnp.maximum(a[:, 0, :], a[:, 1, :])
    a = m.reshape(8, 2, 8, c2)
    h2 = jnp.maximum(a[:, 0, :, :], a[:, 1, :, :]).reshape(64, c2)
    o_ref[...] = jnp.transpose(h2)


def _head_kernel(h_ref, fw_ref, fb_ref, hw_ref, hb_ref, *o_refs):
    z = jnp.dot(h_ref[...], fw_ref[...], preferred_element_type=jnp.float32,
                 precision=jax.lax.Precision.HIGHEST)
    z = jnp.maximum(z + fb_ref[...], 0.0)
    y = jnp.dot(z, hw_ref[...],
                preferred_element_type=jnp.float32,
                precision=jax.lax.Precision.HIGHEST) + hb_ref[...]
    n = y.shape[1] // len(o_refs)
    for t, o_ref in enumerate(o_refs):
        o_ref[...] = y[:, t * n:(t + 1) * n]


def _col_mask(hw, width, lanes, dj):
    j = np.arange(hw) % width
    valid = (j >= 1) if dj < 0 else (j < width - 1)
    return np.repeat(valid[:, None], lanes, axis=1).astype(np.float32)


def _block_diag_taps(w, grp):
    t, cin, cout = w.shape
    eye = jnp.eye(grp, dtype=w.dtype)
    bd = eye[None, :, None, :, None] * w[:, None, :, None, :]
    return bd.reshape(t * grp * cin, grp * cout)


def kernel(x, c1_w, c1_b, c2_w, c2_b, fc1_w, fc1_b, head_w, head_b,
           scale1, shift1, scale2, shift2, scale3, shift3):
    bsz = x.shape[0]
    grp = _GRP
    ngrp = bsz // grp
    size = _SIZE
    npix = size * size

    s1 = scale1[_TASK_ID][None, :]
    sh1 = shift1[_TASK_ID][None, :]
    s2 = scale2[_TASK_ID][None, :]
    sh2 = shift2[_TASK_ID][None, :]
    s3 = scale3[_TASK_ID][None, :]
    sh3 = shift3[_TASK_ID][None, :]

    w1bd = _block_diag_taps(c1_w * s1[0][None, None, :], 8)
    b1 = jnp.tile(s1 * c1_b + sh1, (1, grp))
    w2bd = _block_diag_taps(c2_w * s2[0][None, None, :], 8)
    b2 = jnp.tile(s2 * c2_b + sh2, (1, grp))
    fw = (fc1_w * s3).reshape(64, 32, tdim_fc := fc1_w.shape[1])
    fw = fw.transpose(1, 0, 2).reshape(64 * 32, tdim_fc)
    fb = s3 * fc1_b + sh3

    m1m = _col_mask(npix, size, 3 * grp, -1)
    m1p = _col_mask(npix, size, 3 * grp, 1)
    m2m = _col_mask(npix // 4, size // 2, 16 * grp, -1)
    m2p = _col_mask(npix // 4, size // 2, 16 * grp, 1)

    xg = x.reshape(ngrp, grp, 3, npix).transpose(0, 3, 1, 2)
    xg = xg.reshape(ngrp * npix, grp * 3)

    const = lambda shape: pl.BlockSpec(shape, lambda g: (0, 0))
    h2 = pl.pallas_call(
        functools.partial(_feat_kernel, grp=grp),
        out_shape=jax.ShapeDtypeStruct((ngrp * 32 * grp, 64), jnp.float32),
        grid=(ngrp,),
        in_specs=[
            pl.BlockSpec((npix, grp * 3), lambda g: (g, 0)),
            const(w1bd.shape), const(b1.shape),
            const(m1m.shape), const(m1p.shape),
            const(w2bd.shape), const(b2.shape),
            const(m2m.shape), const(m2p.shape),
        ],
        out_specs=pl.BlockSpec((32 * grp, 64), lambda g: (g, 0)),
        compiler_params=pltpu.CompilerParams(
            dimension_semantics=("parallel",)),
    )(xg, w1bd, b1, m1m, m1p, w2bd, b2, m2m, m2p)

    hflat = h2.reshape(bsz, 64 * 32)

    bm = 256 if bsz % 256 == 0 else bsz
    tdim = head_w.shape[1]
    n = tdim // _NUM_TASKS
    outs = pl.pallas_call(
        _head_kernel,
        out_shape=[jax.ShapeDtypeStruct((bsz, n), jnp.float32)
                   for _ in range(_NUM_TASKS)],
        grid=(bsz // bm,),
        in_specs=[
            pl.BlockSpec((bm, fw.shape[0]), lambda g: (g, 0)),
            const(fw.shape), const(fb.shape),
            const(head_w.shape), const(head_b.shape),
        ],
        out_specs=[pl.BlockSpec((bm, n), lambda g: (g, 0))
                   for _ in range(_NUM_TASKS)],
        compiler_params=pltpu.CompilerParams(
            dimension_semantics=("parallel",)),
    )(hflat, fw, fb, head_w, head_b)
    return list(outs)
